# baseline (speedup 1.0000x reference)
import functools

import jax
import jax.numpy as jnp
from jax import lax
from jax.experimental import pallas as pl
from jax.experimental.pallas import tpu as pltpu
from jax.experimental.pallas import tpu_sc as plsc

NC = 2
NS = 16
NW = NC * NS
CH = 128
ZROWS = 64


NB = 2
NIX = 4


def _sc_edge_make(n_pad, e_pad):
  epw = e_pad // NW
  nchunk = epw // CH
  rpt = n_pad // NS
  mesh = plsc.VectorSubcoreMesh(
      core_axis_name="c", subcore_axis_name="s", num_cores=NC,
      num_subcores=NS)

  scratch = ([
      pltpu.VMEM((NIX, CH), jnp.int32),
      pltpu.VMEM((NIX, CH), jnp.int32),
  ] + [pltpu.VMEM((CH, 128), jnp.float32) for _ in range(NB)] + [
      pltpu.VMEM((ZROWS, 128), jnp.float32),
      pltpu.VMEM_SHARED((n_pad, 128), jnp.float32),
  ] + [pltpu.SemaphoreType.DMA] * (3 * NB))

  @functools.partial(
      pl.kernel,
      out_type=jax.ShapeDtypeStruct((NC * n_pad, 128), jnp.float32),
      mesh=mesh,
      scratch_types=scratch,
  )
  def k(tab_hbm, src_hbm, dst_hbm, out_hbm, sidx, didx, *bufs):
    rows = list(bufs[:NB])
    zb_v = bufs[NB]
    acc_sh = bufs[NB + 1]
    gsem = list(bufs[NB + 2:NB + 2 + NB])
    ssem = list(bufs[NB + 2 + NB:NB + 2 + 2 * NB])
    isem = list(bufs[NB + 2 + 2 * NB:NB + 2 + 3 * NB])
    c = lax.axis_index("c")
    s = lax.axis_index("s")
    wid = s * NC + c
    base = wid * nchunk

    def load_idx(j, slot, b):
      off = pl.multiple_of((base + j) * CH, 8)
      pltpu.async_copy(src_hbm.at[pl.ds(off, CH)], sidx.at[slot], isem[b])
      pltpu.async_copy(dst_hbm.at[pl.ds(off, CH)], didx.at[slot], isem[b])

    def wait_idx(slot, b):
      pltpu.make_async_copy(src_hbm.at[pl.ds(0, CH)], sidx.at[slot],
                            isem[b]).wait()
      pltpu.make_async_copy(src_hbm.at[pl.ds(0, CH)], didx.at[slot],
                            isem[b]).wait()

    for b in range(NB):
      load_idx(b, b, b)

    def zb_body(i, _):
      zb_v[i // 8, pl.ds((i % 8) * 16, 16)] = jnp.zeros((16,), jnp.float32)
      return 0

    lax.fori_loop(0, ZROWS * 8, zb_body, 0)
    for i in range(rpt // ZROWS):
      pltpu.async_copy(zb_v, acc_sh.at[pl.ds(s * rpt + i * ZROWS, ZROWS)],
                       ssem[i % NB])
    for i in range(rpt // ZROWS):
      pltpu.make_async_copy(
          zb_v, acc_sh.at[pl.ds(s * rpt, ZROWS)], ssem[i % NB]).wait()
    plsc.subcore_barrier()

    ngrp = nchunk // NB

    def grp(g, _):
      for b in range(NB):
        j = g * NB + b
        slot = lax.rem(j, NIX)
        wait_idx(slot, b)

        @pl.when(g > 0)
        def _():
          pltpu.make_async_copy(rows[0], acc_sh.at[didx.at[0]],
                                ssem[b]).wait()

        pltpu.async_copy(tab_hbm.at[sidx.at[slot]], rows[b], gsem[b])
        pltpu.make_async_copy(tab_hbm.at[sidx.at[0]], rows[b],
                              gsem[b]).wait()
        pltpu.async_copy(rows[b], acc_sh.at[didx.at[slot]], ssem[b],
                         add=True)

        @pl.when(j + NB < nchunk)
        def _():
          load_idx(j + NB, lax.rem(j + NB, NIX), b)

      return 0

    lax.fori_loop(0, ngrp, grp, 0)
    for b in range(NB):
      pltpu.make_async_copy(rows[0], acc_sh.at[didx.at[0]],
                            ssem[b]).wait()
    plsc.subcore_barrier()

    for i in range(rpt // CH):
      r0 = s * rpt + i * CH
      b = i % NB
      if i >= NB:
        pltpu.make_async_copy(rows[b], out_hbm.at[pl.ds(0, CH)],
                              gsem[b]).wait()
      pltpu.sync_copy(acc_sh.at[pl.ds(r0, CH)], rows[b])
      pltpu.async_copy(rows[b], out_hbm.at[pl.ds(c * n_pad + r0, CH)],
                       gsem[b])
    for b in range(min(NB, rpt // CH)):
      pltpu.make_async_copy(rows[b], out_hbm.at[pl.ds(0, CH)],
                            gsem[b]).wait()

  return k


def _sc_scatter_make(n_nodes, n_pad, feat, e_pad, gather):
  epw = e_pad // NW
  nchunk = epw // CH
  rpt = n_pad // NS
  pk = 128 // feat
  mesh = plsc.VectorSubcoreMesh(
      core_axis_name="c", subcore_axis_name="s", num_cores=NC,
      num_subcores=NS)

  scratch = [
      pltpu.VMEM((CH,), jnp.int32),
      pltpu.VMEM((CH,), jnp.int32),
      pltpu.VMEM((CH, 128), jnp.float32),
      pltpu.VMEM((CH // pk, 128), jnp.float32),
      pltpu.VMEM((ZROWS, 128), jnp.float32),
      pltpu.VMEM_SHARED((n_pad, 128), jnp.float32),
      pltpu.SemaphoreType.DMA,
  ]

  @functools.partial(
      pl.kernel,
      out_type=jax.ShapeDtypeStruct((NC * n_pad, 128), jnp.float32),
      mesh=mesh,
      scratch_types=scratch,
  )
  def k(tab_hbm, src_hbm, dst_hbm, out_hbm, src_v, dst_v, rows_v, pk_v,
        zb_v, acc_sh, sem):
    c = lax.axis_index("c")
    s = lax.axis_index("s")
    wid = s * NC + c

    def zb_body(i, _):
      zb_v[i // 8, pl.ds((i % 8) * 16, 16)] = jnp.zeros((16,), jnp.float32)
      return 0

    lax.fori_loop(0, ZROWS * 8, zb_body, 0)
    if not gather:
      def rz_body(i, _):
        rows_v[i // 8, pl.ds((i % 8) * 16, 16)] = jnp.zeros((16,),
                                                            jnp.float32)
        return 0

      lax.fori_loop(0, CH * 8, rz_body, 0)
    for i in range(rpt // ZROWS):
      pltpu.sync_copy(zb_v, acc_sh.at[pl.ds(s * rpt + i * ZROWS, ZROWS)])
    plsc.subcore_barrier()

    base = wid * epw

    def body(j, _):
      off = base + j * CH
      pltpu.sync_copy(dst_hbm.at[pl.ds(off, CH)], dst_v)
      if gather:
        pltpu.sync_copy(src_hbm.at[pl.ds(off, CH)], src_v)
        pltpu.async_copy(tab_hbm.at[src_v], rows_v, sem).wait()
      else:
        pltpu.sync_copy(
            tab_hbm.at[pl.ds(pl.multiple_of(off // pk, 8), CH // pk)],
            pk_v)

        def exp_body(r, _):
          for v in range(feat // 16):
            rows_v[r, pl.ds(v * 16, 16)] = pk_v[r // pk,
                                                pl.ds((r % pk) * feat
                                                      + v * 16, 16)]
          return 0

        lax.fori_loop(0, CH, exp_body, 0)
      pltpu.sync_copy(rows_v, acc_sh.at[dst_v], add=True)
      return 0

    lax.fori_loop(0, nchunk, body, 0)
    plsc.subcore_barrier()

    for i in range(rpt // CH):
      r0 = s * rpt + i * CH
      pltpu.sync_copy(acc_sh.at[pl.ds(r0, CH)], rows_v)
      pltpu.sync_copy(rows_v, out_hbm.at[pl.ds(c * n_pad + r0, CH)])

  return k


def _dot(a, b):
  return jnp.dot(a, b, preferred_element_type=jnp.float32)


def _embed_body(x_ref, w1, b1, w2, b2, wm, bm, h_ref, hw_ref):
  t = jnp.maximum(_dot(x_ref[...], w1[...]) + b1[...], 0.0)
  h = _dot(t, w2[...]) + b2[...]
  h_ref[...] = h
  hw_ref[...] = _dot(h, wm[...]) + bm[...]


def _update_body(h_ref, a0, a1, e0, e1, wme, ws, bs, wm, bm, h2_ref,
                 hw2_ref):
  agg = a0[...] + a1[...] + _dot(e0[...] + e1[...], wme[...])
  h2 = jnp.maximum(_dot(h_ref[...], ws[...]) + bs[...] + agg, 0.0)
  h2_ref[...] = h2
  hw2_ref[...] = _dot(h2, wm[...]) + bm[...]


def _final_body(h_ref, a0, a1, e0, e1, wme, ws, bs, w1, b1, w2, b2,
                o_ref):
  agg = a0[...] + a1[...] + _dot(e0[...] + e1[...], wme[...])
  h2 = jnp.maximum(_dot(h_ref[...], ws[...]) + bs[...] + agg, 0.0)
  t = jnp.maximum(_dot(h2, w1[...]) + b1[...], 0.0)
  o_ref[...] = _dot(t, w2[...]) + b2[...]


def _f32(*shapes):
  return tuple(jax.ShapeDtypeStruct(s, jnp.float32) for s in shapes)


def kernel(x, edge_index, edge_attr, eW1, eb1, eW2, eb2, cWs, cbs, cWm,
           cbm, hW1, hb1, hW2, hb2):
  n, h_dim = x.shape
  e = edge_index.shape[1]
  ed = edge_attr.shape[1]
  l_layers = cWs.shape[0]
  out_dim = hW2.shape[1]

  n_pad = ((n + NS * ZROWS) // (NS * ZROWS)) * (NS * ZROWS)
  egrain = NW * CH * NB
  e_pad = ((e + egrain - 1) // egrain) * egrain

  src = edge_index[0].astype(jnp.int32)
  dst = edge_index[1].astype(jnp.int32)
  src_p = jnp.concatenate([src, jnp.zeros((e_pad - e,), jnp.int32)])
  dst_p = jnp.concatenate(
      [dst, jnp.full((e_pad - e,), n, jnp.int32)])
  ea_p = jnp.concatenate(
      [edge_attr, jnp.zeros((e_pad - e, ed), jnp.float32)])

  wmh = cWm[:, :h_dim, :]
  wme = cWm[:, h_dim:, :]
  b = lambda v: v.reshape(1, -1)

  sc_edge = _sc_edge_make(n_pad, e_pad)
  sc_ea = _sc_scatter_make(n, n_pad, ed, e_pad, gather=False)

  halves = lambda a: (a[:n], a[n_pad:n_pad + n])

  ea8 = ea_p.reshape(e_pad // (128 // ed), 128)
  ea_out = sc_ea(ea8, src_p, dst_p)
  ea0, ea1 = ea_out[:n, :ed], ea_out[n_pad:n_pad + n, :ed]

  h, hw = pl.pallas_call(
      _embed_body, out_shape=_f32((n, h_dim), (n, h_dim)))(
          x, eW1, b(eb1), eW2, b(eb2), wmh[0], b(cbm[0]))

  hw, ea0, ea1 = lax.optimization_barrier((hw, ea0, ea1))

  for l in range(l_layers - 1):
    a0, a1 = halves(sc_edge(hw, src_p, dst_p))
    h, hw = pl.pallas_call(
        _update_body, out_shape=_f32((n, h_dim), (n, h_dim)))(
            h, a0, a1, ea0, ea1, wme[l], cWs[l], b(cbs[l]),
            wmh[l + 1], b(cbm[l + 1]))

  a0, a1 = halves(sc_edge(hw, src_p, dst_p))
  w2p = jnp.zeros((h_dim, 128), jnp.float32).at[:, :out_dim].set(hW2)
  b2p = jnp.zeros((1, 128), jnp.float32).at[0, :out_dim].set(hb2)
  out = pl.pallas_call(
      _final_body, out_shape=jax.ShapeDtypeStruct((n, 128), jnp.float32))(
          h, a0, a1, ea0, ea1, wme[l_layers - 1],
          cWs[l_layers - 1], b(cbs[l_layers - 1]), hW1, b(hb1), w2p, b2p)
  return out[:, :out_dim]


# speedup vs baseline: 1.0007x; 1.0007x over previous
import functools

import jax
import jax.numpy as jnp
from jax import lax
from jax.experimental import pallas as pl
from jax.experimental.pallas import tpu as pltpu
from jax.experimental.pallas import tpu_sc as plsc

NC = 2
NS = 16
NW = NC * NS
CH = 128
ZROWS = 64


NB = 2
NIX = 4


def _sc_edge_make(n_pad, e_pad):
  epw = e_pad // NW
  nchunk = epw // CH
  rpt = n_pad // NS
  mesh = plsc.VectorSubcoreMesh(
      core_axis_name="c", subcore_axis_name="s", num_cores=NC,
      num_subcores=NS)

  scratch = ([
      pltpu.VMEM((NIX, CH), jnp.int32),
      pltpu.VMEM((NIX, CH), jnp.int32),
  ] + [pltpu.VMEM((CH, 128), jnp.float32) for _ in range(NB)] + [
      pltpu.VMEM((ZROWS, 128), jnp.float32),
      pltpu.VMEM_SHARED((n_pad, 128), jnp.float32),
  ] + [pltpu.SemaphoreType.DMA] * (2 * NB + NIX))

  @functools.partial(
      pl.kernel,
      out_type=jax.ShapeDtypeStruct((NC * n_pad, 128), jnp.float32),
      mesh=mesh,
      scratch_types=scratch,
  )
  def k(tab_hbm, src_hbm, dst_hbm, out_hbm, sidx, didx, *bufs):
    rows = list(bufs[:NB])
    zb_v = bufs[NB]
    acc_sh = bufs[NB + 1]
    gsem = list(bufs[NB + 2:NB + 2 + NB])
    ssem = list(bufs[NB + 2 + NB:NB + 2 + 2 * NB])
    isem = list(bufs[NB + 2 + 2 * NB:NB + 2 + 2 * NB + NIX])
    c = lax.axis_index("c")
    s = lax.axis_index("s")
    wid = s * NC + c
    base = wid * nchunk

    def load_idx(j, slot):
      off = pl.multiple_of((base + j) * CH, 8)
      pltpu.async_copy(src_hbm.at[pl.ds(off, CH)], sidx.at[slot],
                       isem[slot])
      pltpu.async_copy(dst_hbm.at[pl.ds(off, CH)], didx.at[slot],
                       isem[slot])

    def wait_idx(slot):
      pltpu.make_async_copy(src_hbm.at[pl.ds(0, CH)], sidx.at[slot],
                            isem[slot]).wait()
      pltpu.make_async_copy(src_hbm.at[pl.ds(0, CH)], didx.at[slot],
                            isem[slot]).wait()

    for kk in range(min(NIX - 1, nchunk)):
      load_idx(kk, kk)

    def zb_body(i, _):
      zb_v[i // 8, pl.ds((i % 8) * 16, 16)] = jnp.zeros((16,), jnp.float32)
      return 0

    lax.fori_loop(0, ZROWS * 8, zb_body, 0)
    for i in range(rpt // ZROWS):
      pltpu.async_copy(zb_v, acc_sh.at[pl.ds(s * rpt + i * ZROWS, ZROWS)],
                       ssem[i % NB])
    for i in range(rpt // ZROWS):
      pltpu.make_async_copy(
          zb_v, acc_sh.at[pl.ds(s * rpt, ZROWS)], ssem[i % NB]).wait()
    plsc.subcore_barrier()

    ngrp = nchunk // NIX

    wait_idx(0)
    pltpu.async_copy(tab_hbm.at[sidx.at[0]], rows[0], gsem[0])

    def grp(g, _):
      for u in range(NIX):
        j = g * NIX + u
        b = u % NB
        nslot = (u + 1) % NIX

        @pl.when(j + 1 < nchunk)
        def _():
          wait_idx(nslot)

        pltpu.make_async_copy(tab_hbm.at[sidx.at[0]], rows[b],
                              gsem[b]).wait()
        pltpu.async_copy(rows[b], acc_sh.at[didx.at[u]], ssem[b],
                         add=True)

        @pl.when(j > 0)
        def _():
          pltpu.make_async_copy(rows[0], acc_sh.at[didx.at[0]],
                                ssem[(b + 1) % NB]).wait()

        @pl.when(j + 1 < nchunk)
        def _():
          pltpu.async_copy(tab_hbm.at[sidx.at[nslot]], rows[(b + 1) % NB],
                           gsem[(b + 1) % NB])

        @pl.when(j + NIX - 1 < nchunk)
        def _():
          load_idx(j + NIX - 1, (u + NIX - 1) % NIX)

      return 0

    lax.fori_loop(0, ngrp, grp, 0)
    pltpu.make_async_copy(rows[0], acc_sh.at[didx.at[0]],
                          ssem[(nchunk - 1) % NB]).wait()
    plsc.subcore_barrier()

    for i in range(rpt // CH):
      r0 = s * rpt + i * CH
      b = i % NB
      if i >= NB:
        pltpu.make_async_copy(rows[b], out_hbm.at[pl.ds(0, CH)],
                              gsem[b]).wait()
      pltpu.sync_copy(acc_sh.at[pl.ds(r0, CH)], rows[b])
      pltpu.async_copy(rows[b], out_hbm.at[pl.ds(c * n_pad + r0, CH)],
                       gsem[b])
    for b in range(min(NB, rpt // CH)):
      pltpu.make_async_copy(rows[b], out_hbm.at[pl.ds(0, CH)],
                            gsem[b]).wait()

  return k


def _sc_scatter_make(n_nodes, n_pad, feat, e_pad, gather):
  epw = e_pad // NW
  nchunk = epw // CH
  rpt = n_pad // NS
  pk = 128 // feat
  mesh = plsc.VectorSubcoreMesh(
      core_axis_name="c", subcore_axis_name="s", num_cores=NC,
      num_subcores=NS)

  scratch = [
      pltpu.VMEM((CH,), jnp.int32),
      pltpu.VMEM((CH,), jnp.int32),
      pltpu.VMEM((CH, 128), jnp.float32),
      pltpu.VMEM((CH // pk, 128), jnp.float32),
      pltpu.VMEM((ZROWS, 128), jnp.float32),
      pltpu.VMEM_SHARED((n_pad, 128), jnp.float32),
      pltpu.SemaphoreType.DMA,
  ]

  @functools.partial(
      pl.kernel,
      out_type=jax.ShapeDtypeStruct((NC * n_pad, 128), jnp.float32),
      mesh=mesh,
      scratch_types=scratch,
  )
  def k(tab_hbm, src_hbm, dst_hbm, out_hbm, src_v, dst_v, rows_v, pk_v,
        zb_v, acc_sh, sem):
    c = lax.axis_index("c")
    s = lax.axis_index("s")
    wid = s * NC + c

    def zb_body(i, _):
      zb_v[i // 8, pl.ds((i % 8) * 16, 16)] = jnp.zeros((16,), jnp.float32)
      return 0

    lax.fori_loop(0, ZROWS * 8, zb_body, 0)
    if not gather:
      def rz_body(i, _):
        rows_v[i // 8, pl.ds((i % 8) * 16, 16)] = jnp.zeros((16,),
                                                            jnp.float32)
        return 0

      lax.fori_loop(0, CH * 8, rz_body, 0)
    for i in range(rpt // ZROWS):
      pltpu.sync_copy(zb_v, acc_sh.at[pl.ds(s * rpt + i * ZROWS, ZROWS)])
    plsc.subcore_barrier()

    base = wid * epw

    def body(j, _):
      off = base + j * CH
      pltpu.sync_copy(dst_hbm.at[pl.ds(off, CH)], dst_v)
      if gather:
        pltpu.sync_copy(src_hbm.at[pl.ds(off, CH)], src_v)
        pltpu.async_copy(tab_hbm.at[src_v], rows_v, sem).wait()
      else:
        pltpu.sync_copy(
            tab_hbm.at[pl.ds(pl.multiple_of(off // pk, 8), CH // pk)],
            pk_v)

        def exp_body(r, _):
          for v in range(feat // 16):
            rows_v[r, pl.ds(v * 16, 16)] = pk_v[r // pk,
                                                pl.ds((r % pk) * feat
                                                      + v * 16, 16)]
          return 0

        lax.fori_loop(0, CH, exp_body, 0)
      pltpu.sync_copy(rows_v, acc_sh.at[dst_v], add=True)
      return 0

    lax.fori_loop(0, nchunk, body, 0)
    plsc.subcore_barrier()

    for i in range(rpt // CH):
      r0 = s * rpt + i * CH
      pltpu.sync_copy(acc_sh.at[pl.ds(r0, CH)], rows_v)
      pltpu.sync_copy(rows_v, out_hbm.at[pl.ds(c * n_pad + r0, CH)])

  return k


def _dot(a, b):
  return jnp.dot(a, b, preferred_element_type=jnp.float32)


def _embed_body(x_ref, w1, b1, w2, b2, wm, bm, h_ref, hw_ref):
  t = jnp.maximum(_dot(x_ref[...], w1[...]) + b1[...], 0.0)
  h = _dot(t, w2[...]) + b2[...]
  h_ref[...] = h
  hw_ref[...] = _dot(h, wm[...]) + bm[...]


def _update_body(h_ref, a0, a1, e0, e1, wme, ws, bs, wm, bm, h2_ref,
                 hw2_ref):
  agg = a0[...] + a1[...] + _dot(e0[...] + e1[...], wme[...])
  h2 = jnp.maximum(_dot(h_ref[...], ws[...]) + bs[...] + agg, 0.0)
  h2_ref[...] = h2
  hw2_ref[...] = _dot(h2, wm[...]) + bm[...]


def _final_body(h_ref, a0, a1, e0, e1, wme, ws, bs, w1, b1, w2, b2,
                o_ref):
  agg = a0[...] + a1[...] + _dot(e0[...] + e1[...], wme[...])
  h2 = jnp.maximum(_dot(h_ref[...], ws[...]) + bs[...] + agg, 0.0)
  t = jnp.maximum(_dot(h2, w1[...]) + b1[...], 0.0)
  o_ref[...] = _dot(t, w2[...]) + b2[...]


def _f32(*shapes):
  return tuple(jax.ShapeDtypeStruct(s, jnp.float32) for s in shapes)


def kernel(x, edge_index, edge_attr, eW1, eb1, eW2, eb2, cWs, cbs, cWm,
           cbm, hW1, hb1, hW2, hb2):
  n, h_dim = x.shape
  e = edge_index.shape[1]
  ed = edge_attr.shape[1]
  l_layers = cWs.shape[0]
  out_dim = hW2.shape[1]

  n_pad = ((n + NS * ZROWS) // (NS * ZROWS)) * (NS * ZROWS)
  egrain = NW * CH * NIX
  e_pad = ((e + egrain - 1) // egrain) * egrain

  src = edge_index[0].astype(jnp.int32)
  dst = edge_index[1].astype(jnp.int32)
  src_p = jnp.concatenate([src, jnp.zeros((e_pad - e,), jnp.int32)])
  dst_p = jnp.concatenate(
      [dst, jnp.full((e_pad - e,), n, jnp.int32)])
  ea_p = jnp.concatenate(
      [edge_attr, jnp.zeros((e_pad - e, ed), jnp.float32)])

  wmh = cWm[:, :h_dim, :]
  wme = cWm[:, h_dim:, :]
  b = lambda v: v.reshape(1, -1)

  sc_edge = _sc_edge_make(n_pad, e_pad)
  sc_ea = _sc_scatter_make(n, n_pad, ed, e_pad, gather=False)

  halves = lambda a: (a[:n], a[n_pad:n_pad + n])

  ea8 = ea_p.reshape(e_pad // (128 // ed), 128)
  ea_out = sc_ea(ea8, src_p, dst_p)
  ea0, ea1 = ea_out[:n, :ed], ea_out[n_pad:n_pad + n, :ed]

  h, hw = pl.pallas_call(
      _embed_body, out_shape=_f32((n, h_dim), (n, h_dim)))(
          x, eW1, b(eb1), eW2, b(eb2), wmh[0], b(cbm[0]))

  hw, ea0, ea1 = lax.optimization_barrier((hw, ea0, ea1))

  for l in range(l_layers - 1):
    a0, a1 = halves(sc_edge(hw, src_p, dst_p))
    h, hw = pl.pallas_call(
        _update_body, out_shape=_f32((n, h_dim), (n, h_dim)))(
            h, a0, a1, ea0, ea1, wme[l], cWs[l], b(cbs[l]),
            wmh[l + 1], b(cbm[l + 1]))

  a0, a1 = halves(sc_edge(hw, src_p, dst_p))
  w2p = jnp.zeros((h_dim, 128), jnp.float32).at[:, :out_dim].set(hW2)
  b2p = jnp.zeros((1, 128), jnp.float32).at[0, :out_dim].set(hb2)
  out = pl.pallas_call(
      _final_body, out_shape=jax.ShapeDtypeStruct((n, 128), jnp.float32))(
          h, a0, a1, ea0, ea1, wme[l_layers - 1],
          cWs[l_layers - 1], b(cbs[l_layers - 1]), hW1, b(hb1), w2p, b2p)
  return out[:, :out_dim]


# speedup vs baseline: 1.0780x; 1.0773x over previous
import functools

import jax
import jax.numpy as jnp
from jax import lax
from jax.experimental import pallas as pl
from jax.experimental.pallas import tpu as pltpu
from jax.experimental.pallas import tpu_sc as plsc

NC = 2
NS = 16
NW = NC * NS
CH = 128
ZROWS = 64


NB = 2
NIX = 4


def _sc_edge_make(n_pad, e_pad):
  nchunk_tot = e_pad // (CH * NS)
  n0 = (int(nchunk_tot * 3 // 4) // NIX) * NIX
  n1 = nchunk_tot - n0
  assert n1 % NIX == 0 and n0 > 0 and n1 > 0
  rpt = n_pad // NS
  mesh = plsc.VectorSubcoreMesh(
      core_axis_name="c", subcore_axis_name="s", num_cores=NC,
      num_subcores=NS)

  scratch = ([
      pltpu.VMEM((NIX, CH), jnp.int32),
      pltpu.VMEM((NIX, CH), jnp.int32),
  ] + [pltpu.VMEM((CH, 128), jnp.float32) for _ in range(NB)] + [
      pltpu.VMEM((ZROWS, 128), jnp.float32),
      pltpu.VMEM_SHARED((n_pad, 128), jnp.float32),
  ] + [pltpu.SemaphoreType.DMA] * (2 * NB + NIX))

  @functools.partial(
      pl.kernel,
      out_type=jax.ShapeDtypeStruct((NC * n_pad, 128), jnp.float32),
      mesh=mesh,
      scratch_types=scratch,
  )
  def k(tab_hbm, src_hbm, dst_hbm, out_hbm, sidx, didx, *bufs):
    rows = list(bufs[:NB])
    zb_v = bufs[NB]
    acc_sh = bufs[NB + 1]
    gsem = list(bufs[NB + 2:NB + 2 + NB])
    ssem = list(bufs[NB + 2 + NB:NB + 2 + 2 * NB])
    isem = list(bufs[NB + 2 + 2 * NB:NB + 2 + 2 * NB + NIX])
    c = lax.axis_index("c")
    s = lax.axis_index("s")
    nck = jnp.where(c == 0, n0, n1)
    base = jnp.where(c == 0, s * n0, NS * n0 + s * n1)

    def load_idx(j, slot):
      off = pl.multiple_of((base + j) * CH, 8)
      pltpu.async_copy(src_hbm.at[pl.ds(off, CH)], sidx.at[slot],
                       isem[slot])
      pltpu.async_copy(dst_hbm.at[pl.ds(off, CH)], didx.at[slot],
                       isem[slot])

    def wait_idx(slot):
      pltpu.make_async_copy(src_hbm.at[pl.ds(0, CH)], sidx.at[slot],
                            isem[slot]).wait()
      pltpu.make_async_copy(src_hbm.at[pl.ds(0, CH)], didx.at[slot],
                            isem[slot]).wait()

    for kk in range(NIX - 1):
      load_idx(kk, kk)

    def zb_body(i, _):
      zb_v[i // 8, pl.ds((i % 8) * 16, 16)] = jnp.zeros((16,), jnp.float32)
      return 0

    lax.fori_loop(0, ZROWS * 8, zb_body, 0)
    for i in range(rpt // ZROWS):
      pltpu.async_copy(zb_v, acc_sh.at[pl.ds(s * rpt + i * ZROWS, ZROWS)],
                       ssem[i % NB])
    for i in range(rpt // ZROWS):
      pltpu.make_async_copy(
          zb_v, acc_sh.at[pl.ds(s * rpt, ZROWS)], ssem[i % NB]).wait()
    plsc.subcore_barrier()

    ngrp = nck // NIX

    wait_idx(0)
    pltpu.async_copy(tab_hbm.at[sidx.at[0]], rows[0], gsem[0])

    def grp(g, _):
      for u in range(NIX):
        j = g * NIX + u
        b = u % NB
        nslot = (u + 1) % NIX

        @pl.when(j + 1 < nck)
        def _():
          wait_idx(nslot)

        pltpu.make_async_copy(tab_hbm.at[sidx.at[0]], rows[b],
                              gsem[b]).wait()
        pltpu.async_copy(rows[b], acc_sh.at[didx.at[u]], ssem[b],
                         add=True)

        @pl.when(j > 0)
        def _():
          pltpu.make_async_copy(rows[0], acc_sh.at[didx.at[0]],
                                ssem[(b + 1) % NB]).wait()

        @pl.when(j + 1 < nck)
        def _():
          pltpu.async_copy(tab_hbm.at[sidx.at[nslot]], rows[(b + 1) % NB],
                           gsem[(b + 1) % NB])

        @pl.when(j + NIX - 1 < nck)
        def _():
          load_idx(j + NIX - 1, (u + NIX - 1) % NIX)

      return 0

    lax.fori_loop(0, ngrp, grp, 0)
    pltpu.make_async_copy(rows[0], acc_sh.at[didx.at[0]],
                          ssem[1]).wait()
    plsc.subcore_barrier()

    for i in range(rpt // CH):
      r0 = s * rpt + i * CH
      b = i % NB
      if i >= NB:
        pltpu.make_async_copy(rows[b], out_hbm.at[pl.ds(0, CH)],
                              gsem[b]).wait()
      pltpu.sync_copy(acc_sh.at[pl.ds(r0, CH)], rows[b])
      pltpu.async_copy(rows[b], out_hbm.at[pl.ds(c * n_pad + r0, CH)],
                       gsem[b])
    for b in range(min(NB, rpt // CH)):
      pltpu.make_async_copy(rows[b], out_hbm.at[pl.ds(0, CH)],
                            gsem[b]).wait()

  return k


def _sc_scatter_make(n_nodes, n_pad, feat, e_pad, gather):
  epw = e_pad // NW
  nchunk = epw // CH
  rpt = n_pad // NS
  pk = 128 // feat
  mesh = plsc.VectorSubcoreMesh(
      core_axis_name="c", subcore_axis_name="s", num_cores=NC,
      num_subcores=NS)

  scratch = [
      pltpu.VMEM((CH,), jnp.int32),
      pltpu.VMEM((CH,), jnp.int32),
      pltpu.VMEM((CH, 128), jnp.float32),
      pltpu.VMEM((CH // pk, 128), jnp.float32),
      pltpu.VMEM((ZROWS, 128), jnp.float32),
      pltpu.VMEM_SHARED((n_pad, 128), jnp.float32),
      pltpu.SemaphoreType.DMA,
  ]

  @functools.partial(
      pl.kernel,
      out_type=jax.ShapeDtypeStruct((NC * n_pad, 128), jnp.float32),
      mesh=mesh,
      scratch_types=scratch,
  )
  def k(tab_hbm, src_hbm, dst_hbm, out_hbm, src_v, dst_v, rows_v, pk_v,
        zb_v, acc_sh, sem):
    c = lax.axis_index("c")
    s = lax.axis_index("s")
    wid = s * NC + c

    def zb_body(i, _):
      zb_v[i // 8, pl.ds((i % 8) * 16, 16)] = jnp.zeros((16,), jnp.float32)
      return 0

    lax.fori_loop(0, ZROWS * 8, zb_body, 0)
    if not gather:
      def rz_body(i, _):
        rows_v[i // 8, pl.ds((i % 8) * 16, 16)] = jnp.zeros((16,),
                                                            jnp.float32)
        return 0

      lax.fori_loop(0, CH * 8, rz_body, 0)
    for i in range(rpt // ZROWS):
      pltpu.sync_copy(zb_v, acc_sh.at[pl.ds(s * rpt + i * ZROWS, ZROWS)])
    plsc.subcore_barrier()

    base = wid * epw

    def body(j, _):
      off = base + j * CH
      pltpu.sync_copy(dst_hbm.at[pl.ds(off, CH)], dst_v)
      if gather:
        pltpu.sync_copy(src_hbm.at[pl.ds(off, CH)], src_v)
        pltpu.async_copy(tab_hbm.at[src_v], rows_v, sem).wait()
      else:
        pltpu.sync_copy(
            tab_hbm.at[pl.ds(pl.multiple_of(off // pk, 8), CH // pk)],
            pk_v)

        def exp_body(r, _):
          for v in range(feat // 16):
            rows_v[r, pl.ds(v * 16, 16)] = pk_v[r // pk,
                                                pl.ds((r % pk) * feat
                                                      + v * 16, 16)]
          return 0

        lax.fori_loop(0, CH, exp_body, 0)
      pltpu.sync_copy(rows_v, acc_sh.at[dst_v], add=True)
      return 0

    lax.fori_loop(0, nchunk, body, 0)
    plsc.subcore_barrier()

    for i in range(rpt // CH):
      r0 = s * rpt + i * CH
      pltpu.sync_copy(acc_sh.at[pl.ds(r0, CH)], rows_v)
      pltpu.sync_copy(rows_v, out_hbm.at[pl.ds(c * n_pad + r0, CH)])

  return k


def _dot(a, b):
  return jnp.dot(a, b, preferred_element_type=jnp.float32)


def _embed_body(x_ref, w1, b1, w2, b2, wm, bm, h_ref, hw_ref):
  t = jnp.maximum(_dot(x_ref[...], w1[...]) + b1[...], 0.0)
  h = _dot(t, w2[...]) + b2[...]
  h_ref[...] = h
  hw_ref[...] = _dot(h, wm[...]) + bm[...]


def _update_body(h_ref, a0, a1, e0, e1, wme, ws, bs, wm, bm, h2_ref,
                 hw2_ref):
  agg = a0[...] + a1[...] + _dot(e0[...] + e1[...], wme[...])
  h2 = jnp.maximum(_dot(h_ref[...], ws[...]) + bs[...] + agg, 0.0)
  h2_ref[...] = h2
  hw2_ref[...] = _dot(h2, wm[...]) + bm[...]


def _final_body(h_ref, a0, a1, e0, e1, wme, ws, bs, w1, b1, w2, b2,
                o_ref):
  agg = a0[...] + a1[...] + _dot(e0[...] + e1[...], wme[...])
  h2 = jnp.maximum(_dot(h_ref[...], ws[...]) + bs[...] + agg, 0.0)
  t = jnp.maximum(_dot(h2, w1[...]) + b1[...], 0.0)
  o_ref[...] = _dot(t, w2[...]) + b2[...]


def _f32(*shapes):
  return tuple(jax.ShapeDtypeStruct(s, jnp.float32) for s in shapes)


def kernel(x, edge_index, edge_attr, eW1, eb1, eW2, eb2, cWs, cbs, cWm,
           cbm, hW1, hb1, hW2, hb2):
  n, h_dim = x.shape
  e = edge_index.shape[1]
  ed = edge_attr.shape[1]
  l_layers = cWs.shape[0]
  out_dim = hW2.shape[1]

  n_pad = ((n + NS * ZROWS) // (NS * ZROWS)) * (NS * ZROWS)
  egrain = NW * CH * NIX
  e_pad = ((e + egrain - 1) // egrain) * egrain

  src = edge_index[0].astype(jnp.int32)
  dst = edge_index[1].astype(jnp.int32)
  src_p = jnp.concatenate([src, jnp.zeros((e_pad - e,), jnp.int32)])
  dst_p = jnp.concatenate(
      [dst, jnp.full((e_pad - e,), n, jnp.int32)])
  ea_p = jnp.concatenate(
      [edge_attr, jnp.zeros((e_pad - e, ed), jnp.float32)])

  wmh = cWm[:, :h_dim, :]
  wme = cWm[:, h_dim:, :]
  b = lambda v: v.reshape(1, -1)

  sc_edge = _sc_edge_make(n_pad, e_pad)
  sc_ea = _sc_scatter_make(n, n_pad, ed, e_pad, gather=False)

  halves = lambda a: (a[:n], a[n_pad:n_pad + n])

  ea8 = ea_p.reshape(e_pad // (128 // ed), 128)
  ea_out = sc_ea(ea8, src_p, dst_p)
  ea0, ea1 = ea_out[:n, :ed], ea_out[n_pad:n_pad + n, :ed]

  h, hw = pl.pallas_call(
      _embed_body, out_shape=_f32((n, h_dim), (n, h_dim)))(
          x, eW1, b(eb1), eW2, b(eb2), wmh[0], b(cbm[0]))

  hw, ea0, ea1 = lax.optimization_barrier((hw, ea0, ea1))

  for l in range(l_layers - 1):
    a0, a1 = halves(sc_edge(hw, src_p, dst_p))
    h, hw = pl.pallas_call(
        _update_body, out_shape=_f32((n, h_dim), (n, h_dim)))(
            h, a0, a1, ea0, ea1, wme[l], cWs[l], b(cbs[l]),
            wmh[l + 1], b(cbm[l + 1]))

  a0, a1 = halves(sc_edge(hw, src_p, dst_p))
  w2p = jnp.zeros((h_dim, 128), jnp.float32).at[:, :out_dim].set(hW2)
  b2p = jnp.zeros((1, 128), jnp.float32).at[0, :out_dim].set(hb2)
  out = pl.pallas_call(
      _final_body, out_shape=jax.ShapeDtypeStruct((n, 128), jnp.float32))(
          h, a0, a1, ea0, ea1, wme[l_layers - 1],
          cWs[l_layers - 1], b(cbs[l_layers - 1]), hW1, b(hb1), w2p, b2p)
  return out[:, :out_dim]
